# baseline (speedup 1.0000x reference)
import functools

import jax
import jax.numpy as jnp
from jax import lax
from jax.experimental import pallas as pl
from jax.experimental.pallas import tpu as pltpu
from jax.experimental.pallas import tpu_sc as plsc

VOCAB = 1000000
EMB = 32
OUT = 2
SEQ = 200
BATCH = 4096
LANES = 16

NUM_CORES = 2
NUM_SUBCORES = 16
NW = NUM_CORES * NUM_SUBCORES
BPW = BATCH // NW
BS = 8
NSLOT = 2
NBLK = SEQ // BS
INV_SEQ = 1.0 / SEQ


def _embed_pool_body(text_hbm, table_hbm, w_hbm, b_hbm, out_hbm,
                     idx_v, w_v, b_v, buf_v, acc_v, out_t, sems):
    cid = lax.axis_index("c")
    sid = lax.axis_index("s")
    wid = sid * NUM_CORES + cid
    base = wid * BPW

    pltpu.sync_copy(text_hbm.at[:, pl.ds(base, BPW)], idx_v)
    pltpu.sync_copy(w_hbm, w_v)
    pltpu.sync_copy(b_hbm, b_v)

    lane = lax.iota(jnp.int32, LANES)
    bvec = b_v[pl.ds(0, LANES)]

    def fire(c, slot):
        for j in range(BS):
            pltpu.async_copy(table_hbm.at[idx_v.at[c * BS + j]],
                             buf_v.at[slot, j], sems.at[slot])

    def wait(c, slot):
        for j in range(BS):
            pltpu.make_async_copy(table_hbm.at[idx_v.at[c * BS + j]],
                                  buf_v.at[slot, j], sems.at[slot]).wait()

    fire(0, 0)
    fire(1, 1)

    wait(0, 0)

    def init_elem(e, carry):
        for h in range(2):
            v = buf_v[0, 0, e, pl.ds(h * LANES, LANES)]
            for j in range(1, BS):
                v = v + buf_v[0, j, e, pl.ds(h * LANES, LANES)]
            acc_v[pl.ds(e * EMB + h * LANES, LANES)] = v
        return carry

    lax.fori_loop(0, BPW, init_elem, 0, unroll=2)

    def block(c, carry):
        slot = lax.rem(c, NSLOT)

        @pl.when(c + 1 < NBLK)
        def _():
            fire(c + 1, lax.rem(c + 1, NSLOT))

        wait(c, slot)

        def elem(e, carry2):
            for h in range(2):
                v = acc_v[pl.ds(e * EMB + h * LANES, LANES)]
                for j in range(BS):
                    v = v + buf_v[slot, j, e, pl.ds(h * LANES, LANES)]
                acc_v[pl.ds(e * EMB + h * LANES, LANES)] = v
            return carry2

        lax.fori_loop(0, BPW, elem, 0, unroll=2)
        return carry

    lax.fori_loop(1, NBLK, block, 0)

    w_rows = [(w_v[o, pl.ds(0, LANES)], w_v[o, pl.ds(LANES, LANES)])
              for o in range(OUT)]

    def grp(g, carry):
        gbase = g * LANES
        col = (gbase + lane) * EMB
        accs = [jnp.full((LANES,), bvec[o], jnp.float32) for o in range(OUT)]
        for d in range(EMB):
            vals = plsc.load_gather(acc_v, [col + d]) * INV_SEQ
            for o in range(OUT):
                wa, wb = w_rows[o]
                wv = wa[d] if d < LANES else wb[d - LANES]
                accs[o] = accs[o] + vals * wv
        for o in range(OUT):
            out_t[o, pl.ds(gbase, LANES)] = accs[o]
        return carry

    lax.fori_loop(0, BPW // LANES, grp, 0)

    for o in range(OUT):
        pltpu.sync_copy(out_t.at[o], out_hbm.at[o, pl.ds(base, BPW)])


_embed_pool = functools.partial(
    pl.kernel,
    out_type=jax.ShapeDtypeStruct((OUT, BATCH), jnp.float32),
    mesh=plsc.VectorSubcoreMesh(core_axis_name="c", subcore_axis_name="s",
                                num_cores=NUM_CORES,
                                num_subcores=NUM_SUBCORES),
    scratch_types=[
        pltpu.VMEM((SEQ, BPW), jnp.int32),
        pltpu.VMEM((OUT, EMB), jnp.float32),
        pltpu.VMEM((LANES,), jnp.float32),
        pltpu.VMEM((NSLOT, BS, BPW, EMB), jnp.float32),
        pltpu.VMEM((BPW * EMB,), jnp.float32),
        pltpu.VMEM((OUT, BPW), jnp.float32),
        pltpu.SemaphoreType.DMA((NSLOT,)),
    ],
    compiler_params=pltpu.CompilerParams(needs_layout_passes=False,
                                         use_tc_tiling_on_sc=False),
)(_embed_pool_body)


def kernel(text, lengths, table, W, b):
    del lengths
    bpad = jnp.zeros((LANES,), jnp.float32).at[:OUT].set(b)
    return _embed_pool(text, table, W, bpad).T


# speedup vs baseline: 4.4839x; 4.4839x over previous
import functools

import jax
import jax.numpy as jnp
from jax import lax
from jax.experimental import pallas as pl
from jax.experimental.pallas import tpu as pltpu
from jax.experimental.pallas import tpu_sc as plsc

VOCAB = 1000000
EMB = 32
OUT = 2
SEQ = 200
BATCH = 4096
LANES = 16

NUM_CORES = 2
NUM_SUBCORES = 16
NW = NUM_CORES * NUM_SUBCORES
BPW = BATCH // NW
BS = 20
NSLOT = 3
NBLK = SEQ // BS
INV_SEQ = 1.0 / SEQ

FOLD_BLK = 32768


def _fold_body(w_ref, tt_ref, s0_ref, s1_ref):
    out = jnp.dot(w_ref[...], tt_ref[...],
                  preferred_element_type=jnp.float32) * INV_SEQ
    s0_ref[...] = out[0]
    s1_ref[...] = out[1]


_fold = pl.pallas_call(
    _fold_body,
    grid=(pl.cdiv(VOCAB, FOLD_BLK),),
    in_specs=[
        pl.BlockSpec((OUT, EMB), lambda i: (0, 0)),
        pl.BlockSpec((EMB, FOLD_BLK), lambda i: (0, i)),
    ],
    out_specs=[
        pl.BlockSpec((FOLD_BLK,), lambda i: (i,)),
        pl.BlockSpec((FOLD_BLK,), lambda i: (i,)),
    ],
    out_shape=[
        jax.ShapeDtypeStruct((VOCAB,), jnp.float32),
        jax.ShapeDtypeStruct((VOCAB,), jnp.float32),
    ],
)


def _pool_body(text_hbm, s0_hbm, s1_hbm, b_hbm, out_hbm,
               idx_v, b_v, buf_v, out_t, sems):
    cid = lax.axis_index("c")
    sid = lax.axis_index("s")
    wid = sid * NUM_CORES + cid
    base = wid * BPW

    pltpu.sync_copy(text_hbm.at[:, pl.ds(base, BPW)], idx_v)
    pltpu.sync_copy(b_hbm, b_v)
    bvec = b_v[pl.ds(0, LANES)]

    srcs = (s0_hbm, s1_hbm)

    def fire(c, slot):
        for j in range(BS):
            for o in range(OUT):
                pltpu.async_copy(srcs[o].at[idx_v.at[c * BS + j]],
                                 buf_v.at[slot, j, o], sems.at[slot])

    def wait(c, slot):
        for j in range(BS):
            for o in range(OUT):
                pltpu.make_async_copy(srcs[o].at[idx_v.at[c * BS + j]],
                                      buf_v.at[slot, j, o],
                                      sems.at[slot]).wait()

    fire(0, 0)
    fire(1, 1)

    NV = BPW // LANES

    def block(c, carry):
        slot = lax.rem(c, NSLOT)

        @pl.when(c + 2 < NBLK)
        def _():
            fire(c + 2, lax.rem(c + 2, NSLOT))

        wait(c, slot)

        accs = list(carry)
        for j in range(BS):
            for o in range(OUT):
                for k in range(NV):
                    accs[o * NV + k] = (
                        accs[o * NV + k]
                        + buf_v[slot, j, o, pl.ds(k * LANES, LANES)])
        return tuple(accs)

    zeros = jnp.zeros((LANES,), jnp.float32)
    accs = lax.fori_loop(0, NBLK, block, (zeros,) * (OUT * NV))

    for o in range(OUT):
        bo = jnp.full((LANES,), bvec[o], jnp.float32)
        for k in range(NV):
            out_t[o, pl.ds(k * LANES, LANES)] = accs[o * NV + k] + bo

    for o in range(OUT):
        pltpu.sync_copy(out_t.at[o], out_hbm.at[o, pl.ds(base, BPW)])


_pool = functools.partial(
    pl.kernel,
    out_type=jax.ShapeDtypeStruct((OUT, BATCH), jnp.float32),
    mesh=plsc.VectorSubcoreMesh(core_axis_name="c", subcore_axis_name="s",
                                num_cores=NUM_CORES,
                                num_subcores=NUM_SUBCORES),
    scratch_types=[
        pltpu.VMEM((SEQ, BPW), jnp.int32),
        pltpu.VMEM((LANES,), jnp.float32),
        pltpu.VMEM((NSLOT, BS, OUT, BPW), jnp.float32),
        pltpu.VMEM((OUT, BPW), jnp.float32),
        pltpu.SemaphoreType.DMA((NSLOT,)),
    ],
    compiler_params=pltpu.CompilerParams(use_tc_tiling_on_sc=False),
)(_pool_body)


def kernel(text, lengths, table, W, b):
    del lengths
    s0, s1 = _fold(W, table.T)
    bpad = jnp.zeros((LANES,), jnp.float32).at[:OUT].set(b)
    return _pool(text, s0, s1, bpad).T


# speedup vs baseline: 5.7993x; 1.2934x over previous
import functools

import jax
import jax.numpy as jnp
from jax import lax
from jax.experimental import pallas as pl
from jax.experimental.pallas import tpu as pltpu
from jax.experimental.pallas import tpu_sc as plsc

VOCAB = 1000000
EMB = 32
OUT = 2
SEQ = 200
BATCH = 4096
LANES = 16

NUM_CORES = 2
NUM_SUBCORES = 16
NW = NUM_CORES * NUM_SUBCORES
BPW = BATCH // NW
BS = 20
NSLOT = 3
NBLK = SEQ // BS
INV_SEQ = 1.0 / SEQ

FOLD_BLK = 32768


def _fold_body(w_ref, tt_ref, s_ref):
    out = jnp.dot(w_ref[...], tt_ref[...],
                  preferred_element_type=jnp.float32) * INV_SEQ
    bb = out.astype(jnp.bfloat16)
    u = jax.lax.bitcast_convert_type(bb, jnp.uint16).astype(jnp.uint32)
    s_ref[...] = (u[1] << 16) | u[0]


_fold = pl.pallas_call(
    _fold_body,
    grid=(pl.cdiv(VOCAB, FOLD_BLK),),
    in_specs=[
        pl.BlockSpec((OUT, EMB), lambda i: (0, 0)),
        pl.BlockSpec((EMB, FOLD_BLK), lambda i: (0, i)),
    ],
    out_specs=pl.BlockSpec((FOLD_BLK,), lambda i: (i,)),
    out_shape=jax.ShapeDtypeStruct((VOCAB,), jnp.uint32),
)


def _pool_body(text_hbm, s_hbm, b_hbm, out_hbm,
               idx_v, b_v, buf_v, out_t, sems):
    cid = lax.axis_index("c")
    sid = lax.axis_index("s")
    wid = sid * NUM_CORES + cid
    base = wid * BPW

    pltpu.sync_copy(text_hbm.at[:, pl.ds(base, BPW)], idx_v)
    pltpu.sync_copy(b_hbm, b_v)
    bvec = b_v[pl.ds(0, LANES)]

    def fire(c, slot):
        for j in range(BS):
            pltpu.async_copy(s_hbm.at[idx_v.at[c * BS + j]],
                             buf_v.at[slot, j], sems.at[slot])

    def wait(c, slot):
        for j in range(BS):
            pltpu.make_async_copy(s_hbm.at[idx_v.at[c * BS + j]],
                                  buf_v.at[slot, j], sems.at[slot]).wait()

    fire(0, 0)
    fire(1, 1)

    NV = BPW // LANES

    def block(c, carry):
        slot = lax.rem(c, NSLOT)

        @pl.when(c + 2 < NBLK)
        def _():
            fire(c + 2, lax.rem(c + 2, NSLOT))

        wait(c, slot)

        accs = list(carry)
        himask = jnp.full((LANES,), 0xFFFF0000, jnp.uint32)
        for j in range(BS):
            for k in range(NV):
                u = buf_v[slot, j, pl.ds(k * LANES, LANES)]
                f0 = jax.lax.bitcast_convert_type(u << 16, jnp.float32)
                f1 = jax.lax.bitcast_convert_type(u & himask, jnp.float32)
                accs[k] = accs[k] + f0
                accs[NV + k] = accs[NV + k] + f1
        return tuple(accs)

    zeros = jnp.zeros((LANES,), jnp.float32)
    accs = lax.fori_loop(0, NBLK, block, (zeros,) * (OUT * NV))

    for o in range(OUT):
        bo = jnp.full((LANES,), bvec[o], jnp.float32)
        for k in range(NV):
            out_t[o, pl.ds(k * LANES, LANES)] = accs[o * NV + k] + bo

    for o in range(OUT):
        pltpu.sync_copy(out_t.at[o], out_hbm.at[o, pl.ds(base, BPW)])


_pool = functools.partial(
    pl.kernel,
    out_type=jax.ShapeDtypeStruct((OUT, BATCH), jnp.float32),
    mesh=plsc.VectorSubcoreMesh(core_axis_name="c", subcore_axis_name="s",
                                num_cores=NUM_CORES,
                                num_subcores=NUM_SUBCORES),
    scratch_types=[
        pltpu.VMEM((SEQ, BPW), jnp.int32),
        pltpu.VMEM((LANES,), jnp.float32),
        pltpu.VMEM((NSLOT, BS, BPW), jnp.uint32),
        pltpu.VMEM((OUT, BPW), jnp.float32),
        pltpu.SemaphoreType.DMA((NSLOT,)),
    ],
    compiler_params=pltpu.CompilerParams(use_tc_tiling_on_sc=False),
)(_pool_body)


def kernel(text, lengths, table, W, b):
    del lengths
    s_packed = _fold(W, table.T)
    bpad = jnp.zeros((LANES,), jnp.float32).at[:OUT].set(b)
    return _pool(text, s_packed, bpad).T


# speedup vs baseline: 6.0662x; 1.0460x over previous
import functools

import jax
import jax.numpy as jnp
from jax import lax
from jax.experimental import pallas as pl
from jax.experimental.pallas import tpu as pltpu
from jax.experimental.pallas import tpu_sc as plsc

VOCAB = 1000000
EMB = 32
OUT = 2
SEQ = 200
BATCH = 4096
LANES = 16

NUM_CORES = 2
NUM_SUBCORES = 16
NW = NUM_CORES * NUM_SUBCORES
BPW = BATCH // NW
BS = 20
NSLOT = 4
NBLK = SEQ // BS
INV_SEQ = 1.0 / SEQ

FOLD_BLK = 65536


def _fold_body(w_ref, tt_ref, s_ref):
    out = jnp.dot(w_ref[...], tt_ref[...],
                  preferred_element_type=jnp.float32) * INV_SEQ
    bb = out.astype(jnp.bfloat16)
    u = jax.lax.bitcast_convert_type(bb, jnp.uint16).astype(jnp.uint32)
    s_ref[...] = (u[1] << 16) | u[0]


_fold = pl.pallas_call(
    _fold_body,
    grid=(pl.cdiv(VOCAB, FOLD_BLK),),
    in_specs=[
        pl.BlockSpec((OUT, EMB), lambda i: (0, 0)),
        pl.BlockSpec((EMB, FOLD_BLK), lambda i: (0, i)),
    ],
    out_specs=pl.BlockSpec((FOLD_BLK,), lambda i: (i,)),
    out_shape=jax.ShapeDtypeStruct((VOCAB,), jnp.uint32),
)


def _pool_body(text_hbm, s_hbm, b_hbm, out_hbm,
               idx_v, b_v, buf_v, out_t, sems):
    cid = lax.axis_index("c")
    sid = lax.axis_index("s")
    wid = sid * NUM_CORES + cid
    base = wid * BPW

    pltpu.sync_copy(text_hbm.at[:, pl.ds(base, BPW)], idx_v)
    pltpu.sync_copy(b_hbm, b_v)
    bvec = b_v[pl.ds(0, LANES)]

    def fire(c, slot):
        def one(j, carry):
            pltpu.async_copy(s_hbm.at[idx_v.at[c * BS + j]],
                             buf_v.at[slot, j], sems.at[slot])
            return carry
        lax.fori_loop(0, BS, one, 0)

    def wait(c, slot):
        def one(j, carry):
            pltpu.make_async_copy(s_hbm.at[idx_v.at[c * BS + j]],
                                  buf_v.at[slot, j], sems.at[slot]).wait()
            return carry
        lax.fori_loop(0, BS, one, 0)

    fire(0, 0)
    fire(1, 1)
    fire(2, 2)

    NV = BPW // LANES

    def block(c, carry):
        slot = lax.rem(c, NSLOT)

        @pl.when(c + 3 < NBLK)
        def _():
            fire(c + 3, lax.rem(c + 3, NSLOT))

        wait(c, slot)

        himask = jnp.full((LANES,), 0xFFFF0000, jnp.uint32)

        def row(j, accs_t):
            accs2 = list(accs_t)
            for k in range(NV):
                u = buf_v[slot, j, pl.ds(k * LANES, LANES)]
                f0 = jax.lax.bitcast_convert_type(u << 16, jnp.float32)
                f1 = jax.lax.bitcast_convert_type(u & himask, jnp.float32)
                accs2[k] = accs2[k] + f0
                accs2[NV + k] = accs2[NV + k] + f1
            return tuple(accs2)

        return lax.fori_loop(0, BS, row, carry)

    zeros = jnp.zeros((LANES,), jnp.float32)
    accs = lax.fori_loop(0, NBLK, block, (zeros,) * (OUT * NV))

    for o in range(OUT):
        bo = jnp.full((LANES,), bvec[o], jnp.float32)
        for k in range(NV):
            out_t[o, pl.ds(k * LANES, LANES)] = accs[o * NV + k] + bo

    for o in range(OUT):
        pltpu.sync_copy(out_t.at[o], out_hbm.at[o, pl.ds(base, BPW)])


_pool = functools.partial(
    pl.kernel,
    out_type=jax.ShapeDtypeStruct((OUT, BATCH), jnp.float32),
    mesh=plsc.VectorSubcoreMesh(core_axis_name="c", subcore_axis_name="s",
                                num_cores=NUM_CORES,
                                num_subcores=NUM_SUBCORES),
    scratch_types=[
        pltpu.VMEM((SEQ, BPW), jnp.int32),
        pltpu.VMEM((LANES,), jnp.float32),
        pltpu.VMEM((NSLOT, BS, BPW), jnp.uint32),
        pltpu.VMEM((OUT, BPW), jnp.float32),
        pltpu.SemaphoreType.DMA((NSLOT,)),
    ],
    compiler_params=pltpu.CompilerParams(use_tc_tiling_on_sc=False),
)(_pool_body)


def kernel(text, lengths, table, W, b):
    del lengths
    s_packed = _fold(W, table.T)
    bpad = jnp.zeros((LANES,), jnp.float32).at[:OUT].set(b)
    return _pool(text, s_packed, bpad).T


# speedup vs baseline: 6.3919x; 1.0537x over previous
import functools

import jax
import jax.numpy as jnp
from jax import lax
from jax.experimental import pallas as pl
from jax.experimental.pallas import tpu as pltpu
from jax.experimental.pallas import tpu_sc as plsc

VOCAB = 1000000
EMB = 32
OUT = 2
SEQ = 200
BATCH = 4096
LANES = 16

NUM_CORES = 2
NUM_SUBCORES = 16
NW = NUM_CORES * NUM_SUBCORES
BPW = BATCH // NW
BS = 25
NSLOT = 4
NBLK = SEQ // BS
INV_SEQ = 1.0 / SEQ

FOLD_BLK = 65536


def _fold_body(w_ref, tt_ref, s_ref):
    out = jnp.dot(w_ref[...], tt_ref[...],
                  preferred_element_type=jnp.float32) * INV_SEQ
    bb = out.astype(jnp.bfloat16)
    u = jax.lax.bitcast_convert_type(bb, jnp.uint16).astype(jnp.uint32)
    s_ref[...] = (u[1] << 16) | u[0]


_fold = pl.pallas_call(
    _fold_body,
    grid=(pl.cdiv(VOCAB, FOLD_BLK),),
    in_specs=[
        pl.BlockSpec((OUT, EMB), lambda i: (0, 0)),
        pl.BlockSpec((EMB, FOLD_BLK), lambda i: (0, i)),
    ],
    out_specs=pl.BlockSpec((FOLD_BLK,), lambda i: (i,)),
    out_shape=jax.ShapeDtypeStruct((VOCAB,), jnp.uint32),
)


def _pool_body(text_hbm, s_hbm, b_hbm, out_hbm,
               idx_v, b_v, buf_v, out_t, sems):
    cid = lax.axis_index("c")
    sid = lax.axis_index("s")
    wid = sid * NUM_CORES + cid
    base = wid * BPW

    pltpu.sync_copy(text_hbm.at[:, wid], idx_v)
    pltpu.sync_copy(b_hbm, b_v)
    bvec = b_v[pl.ds(0, LANES)]

    def fire(c, slot):
        def one(j, carry):
            r = c * BS + j
            pltpu.async_copy(s_hbm.at[idx_v.at[r // 8, r % 8]],
                             buf_v.at[slot, j], sems.at[slot])
            return carry
        lax.fori_loop(0, BS, one, 0)

    def wait(c, slot):
        def one(j, carry):
            r = c * BS + j
            pltpu.make_async_copy(s_hbm.at[idx_v.at[r // 8, r % 8]],
                                  buf_v.at[slot, j], sems.at[slot]).wait()
            return carry
        lax.fori_loop(0, BS, one, 0)

    fire(0, 0)
    fire(1, 1)
    fire(2, 2)

    NV = BPW // LANES

    def block(c, carry):
        slot = lax.rem(c, NSLOT)

        @pl.when(c + 3 < NBLK)
        def _():
            fire(c + 3, lax.rem(c + 3, NSLOT))

        wait(c, slot)

        himask = jnp.full((LANES,), 0xFFFF0000, jnp.uint32)

        def row(j, accs_t):
            accs2 = list(accs_t)
            for k in range(NV):
                u = buf_v[slot, j, pl.ds(k * LANES, LANES)]
                f0 = jax.lax.bitcast_convert_type(u << 16, jnp.float32)
                f1 = jax.lax.bitcast_convert_type(u & himask, jnp.float32)
                accs2[k] = accs2[k] + f0
                accs2[NV + k] = accs2[NV + k] + f1
            return tuple(accs2)

        return lax.fori_loop(0, BS, row, carry)

    zeros = jnp.zeros((LANES,), jnp.float32)
    accs = lax.fori_loop(0, NBLK, block, (zeros,) * (OUT * NV))

    for o in range(OUT):
        bo = jnp.full((LANES,), bvec[o], jnp.float32)
        for k in range(NV):
            out_t[o, pl.ds(k * LANES, LANES)] = accs[o * NV + k] + bo

    for o in range(OUT):
        pltpu.sync_copy(out_t.at[o], out_hbm.at[o, pl.ds(base, BPW)])


_pool = functools.partial(
    pl.kernel,
    out_type=jax.ShapeDtypeStruct((OUT, BATCH), jnp.float32),
    mesh=plsc.VectorSubcoreMesh(core_axis_name="c", subcore_axis_name="s",
                                num_cores=NUM_CORES,
                                num_subcores=NUM_SUBCORES),
    scratch_types=[
        pltpu.VMEM((SEQ // 8, 8, BPW), jnp.int32),
        pltpu.VMEM((LANES,), jnp.float32),
        pltpu.VMEM((NSLOT, BS, BPW), jnp.uint32),
        pltpu.VMEM((OUT, BPW), jnp.float32),
        pltpu.SemaphoreType.DMA((NSLOT,)),
    ],
    compiler_params=pltpu.CompilerParams(use_tc_tiling_on_sc=False),
)(_pool_body)


def kernel(text, lengths, table, W, b):
    del lengths
    s_packed = _fold(W, table.T)
    bpad = jnp.zeros((LANES,), jnp.float32).at[:OUT].set(b)
    text_tiled = text.reshape(SEQ // 8, 8, BATCH // 128, 128)\
                     .transpose(0, 2, 1, 3)
    return _pool(text_tiled, s_packed, bpad).T
